# baseline (speedup 1.0000x reference)
import jax
import jax.numpy as jnp
from jax.experimental import pallas as pl
from jax.experimental.pallas import tpu as pltpu

N = 4096
IC = 128
F = 64

RB = 512
CB = 512


def _xdw_kernel(x_ref, w1_ref, b1_ref, alpha_ref, a_ref, res_ref):
    xdw = (
        jnp.dot(x_ref[...], w1_ref[...], preferred_element_type=jnp.float32)
        + b1_ref[...]
    )
    al = alpha_ref[0, 0]
    a_ref[...] = al * xdw
    res_ref[...] = (1.0 - al) * xdw


def _main_kernel(
    e0_ref,
    e1_ref,
    a_ref,
    res_ref,
    w2a_ref,
    w2b_ref,
    b2_ref,
    bias_ref,
    out_e_ref,
    out_n_ref,
    acc1_ref,
    acc2_ref,
):
    j = pl.program_id(1)
    nj = pl.num_programs(1)
    e0 = e0_ref[...]
    e1 = e1_ref[...]

    lane = jax.lax.broadcasted_iota(jnp.int32, (RB, 128), 1)
    even = (lane % 2) == 0
    for v in range(2 * CB // 128):
        p, h = divmod(v, 2)
        idx = 64 * h + lane // 2
        src0 = e0[:, 128 * p : 128 * (p + 1)]
        src1 = e1[:, 128 * p : 128 * (p + 1)]
        ga = jnp.take_along_axis(src0, idx, axis=1)
        gb = jnp.take_along_axis(src1, idx, axis=1)
        out_e_ref[:, 128 * v : 128 * (v + 1)] = jnp.where(even, ga, gb)

    @pl.when(j == 0)
    def _():
        acc1_ref[...] = jnp.zeros_like(acc1_ref)
        acc2_ref[...] = jnp.zeros_like(acc2_ref)

    a = a_ref[...]
    acc1_ref[...] += jnp.dot(e0, a, preferred_element_type=jnp.float32)
    acc2_ref[...] += jnp.dot(e1, a, preferred_element_type=jnp.float32)

    @pl.when(j == nj - 1)
    def _():
        resid = res_ref[...]
        s1 = acc1_ref[...] + resid
        s2 = acc2_ref[...] + resid
        h2 = (
            jnp.dot(s1, w2a_ref[...], preferred_element_type=jnp.float32)
            + jnp.dot(s2, w2b_ref[...], preferred_element_type=jnp.float32)
            + b2_ref[...]
            + bias_ref[...]
        )
        out_n_ref[...] = jnp.maximum(h2, 0.0)


def kernel(X, E0, E1, alpha, W1, b1, W2, b2, bias):
    alpha2 = jnp.asarray(alpha, jnp.float32).reshape(1, 1)
    a, res = pl.pallas_call(
        _xdw_kernel,
        out_shape=[
            jax.ShapeDtypeStruct((N, F), jnp.float32),
            jax.ShapeDtypeStruct((N, F), jnp.float32),
        ],
    )(X, W1, b1.reshape(1, F), alpha2)

    grid = (N // RB, N // CB)
    out_e_flat, out_n = pl.pallas_call(
        _main_kernel,
        grid=grid,
        in_specs=[
            pl.BlockSpec((RB, CB), lambda i, j: (i, j)),
            pl.BlockSpec((RB, CB), lambda i, j: (i, j)),
            pl.BlockSpec((CB, F), lambda i, j: (j, 0)),
            pl.BlockSpec((RB, F), lambda i, j: (i, 0)),
            pl.BlockSpec((F, F), lambda i, j: (0, 0)),
            pl.BlockSpec((F, F), lambda i, j: (0, 0)),
            pl.BlockSpec((1, F), lambda i, j: (0, 0)),
            pl.BlockSpec((1, F), lambda i, j: (0, 0)),
        ],
        out_specs=[
            pl.BlockSpec((RB, 2 * CB), lambda i, j: (i, j)),
            pl.BlockSpec((RB, F), lambda i, j: (i, 0)),
        ],
        out_shape=[
            jax.ShapeDtypeStruct((N, 2 * N), jnp.float32),
            jax.ShapeDtypeStruct((N, F), jnp.float32),
        ],
        scratch_shapes=[
            pltpu.VMEM((RB, F), jnp.float32),
            pltpu.VMEM((RB, F), jnp.float32),
        ],
    )(E0, E1, a, res, W2[:F], W2[F:], b2.reshape(1, F), bias.reshape(1, F))

    out_E = out_e_flat.reshape(N, N, 2)
    return (out_n, out_E)


# speedup vs baseline: 1.5400x; 1.5400x over previous
import jax
import jax.numpy as jnp
from jax.experimental import pallas as pl
from jax.experimental.pallas import tpu as pltpu

N = 4096
IC = 128
F = 64

RB = 256
CB = 512


def _xdw_kernel(x_ref, w1_ref, b1_ref, alpha_ref, a_ref, res_ref):
    xdw = (
        jnp.dot(x_ref[...], w1_ref[...], preferred_element_type=jnp.float32)
        + b1_ref[...]
    )
    al = alpha_ref[0, 0]
    a_ref[...] = al * xdw
    res_ref[...] = (1.0 - al) * xdw


def _main_kernel(
    e0_ref,
    e1_ref,
    a_ref,
    res_ref,
    w2a_ref,
    w2b_ref,
    b2_ref,
    bias_ref,
    out_e_ref,
    out_n_ref,
    acc1_ref,
    acc2_ref,
):
    j = pl.program_id(1)
    nj = pl.num_programs(1)
    e0 = e0_ref[...]
    e1 = e1_ref[...]

    out_e_ref[:, pl.ds(j * CB, CB)] = e0
    out_e_ref[:, pl.ds(N + j * CB, CB)] = e1

    @pl.when(j == 0)
    def _():
        acc1_ref[...] = jnp.zeros_like(acc1_ref)
        acc2_ref[...] = jnp.zeros_like(acc2_ref)

    a = a_ref[...]
    acc1_ref[...] += jnp.dot(e0, a, preferred_element_type=jnp.float32)
    acc2_ref[...] += jnp.dot(e1, a, preferred_element_type=jnp.float32)

    @pl.when(j == nj - 1)
    def _():
        resid = res_ref[...]
        s1 = acc1_ref[...] + resid
        s2 = acc2_ref[...] + resid
        h2 = (
            jnp.dot(s1, w2a_ref[...], preferred_element_type=jnp.float32)
            + jnp.dot(s2, w2b_ref[...], preferred_element_type=jnp.float32)
            + b2_ref[...]
            + bias_ref[...]
        )
        out_n_ref[...] = jnp.maximum(h2, 0.0)


def kernel(X, E0, E1, alpha, W1, b1, W2, b2, bias):
    alpha2 = jnp.asarray(alpha, jnp.float32).reshape(1, 1)
    a, res = pl.pallas_call(
        _xdw_kernel,
        out_shape=[
            jax.ShapeDtypeStruct((N, F), jnp.float32),
            jax.ShapeDtypeStruct((N, F), jnp.float32),
        ],
    )(X, W1, b1.reshape(1, F), alpha2)

    grid = (N // RB, N // CB)
    out_e_flat, out_n = pl.pallas_call(
        _main_kernel,
        grid=grid,
        in_specs=[
            pl.BlockSpec((RB, CB), lambda i, j: (i, j)),
            pl.BlockSpec((RB, CB), lambda i, j: (i, j)),
            pl.BlockSpec((CB, F), lambda i, j: (j, 0)),
            pl.BlockSpec((RB, F), lambda i, j: (i, 0)),
            pl.BlockSpec((F, F), lambda i, j: (0, 0)),
            pl.BlockSpec((F, F), lambda i, j: (0, 0)),
            pl.BlockSpec((1, F), lambda i, j: (0, 0)),
            pl.BlockSpec((1, F), lambda i, j: (0, 0)),
        ],
        out_specs=[
            pl.BlockSpec((RB, 2 * N), lambda i, j: (i, 0)),
            pl.BlockSpec((RB, F), lambda i, j: (i, 0)),
        ],
        out_shape=[
            jax.ShapeDtypeStruct((N, 2 * N), jnp.float32),
            jax.ShapeDtypeStruct((N, F), jnp.float32),
        ],
        scratch_shapes=[
            pltpu.VMEM((RB, F), jnp.float32),
            pltpu.VMEM((RB, F), jnp.float32),
        ],
    )(E0, E1, a, res, W2[:F], W2[F:], b2.reshape(1, F), bias.reshape(1, F))

    out_E = out_e_flat.reshape(N, 2, N).swapaxes(1, 2)
    return (out_n, out_E)


# speedup vs baseline: 1.7806x; 1.1562x over previous
import jax
import jax.numpy as jnp
from jax.experimental import pallas as pl
from jax.experimental.pallas import tpu as pltpu

N = 4096
IC = 128
F = 64

RB = 512
CB = 512


def _xdw_kernel(x_ref, w1_ref, b1_ref, alpha_ref, a_ref, res_ref):
    xdw = (
        jnp.dot(x_ref[...], w1_ref[...], preferred_element_type=jnp.float32)
        + b1_ref[...]
    )
    al = alpha_ref[0, 0]
    a_ref[...] = al * xdw
    res_ref[...] = (1.0 - al) * xdw


def _main_kernel(
    e0_ref,
    e1_ref,
    a_ref,
    res_ref,
    w2a_ref,
    w2b_ref,
    b2_ref,
    bias_ref,
    out_e_ref,
    out_n_ref,
    acc1_ref,
    acc2_ref,
):
    j = pl.program_id(1)
    nj = pl.num_programs(1)
    e0 = e0_ref[...]
    e1 = e1_ref[...]

    for t in range(CB // 128):
        out_e_ref[:, 256 * t : 256 * t + 128] = e0[:, 128 * t : 128 * (t + 1)]
        out_e_ref[:, 256 * t + 128 : 256 * t + 256] = e1[:, 128 * t : 128 * (t + 1)]

    @pl.when(j == 0)
    def _():
        acc1_ref[...] = jnp.zeros_like(acc1_ref)
        acc2_ref[...] = jnp.zeros_like(acc2_ref)

    a = a_ref[...]
    acc1_ref[...] += jnp.dot(e0, a, preferred_element_type=jnp.float32)
    acc2_ref[...] += jnp.dot(e1, a, preferred_element_type=jnp.float32)

    @pl.when(j == nj - 1)
    def _():
        resid = res_ref[...]
        s1 = acc1_ref[...] + resid
        s2 = acc2_ref[...] + resid
        h2 = (
            jnp.dot(s1, w2a_ref[...], preferred_element_type=jnp.float32)
            + jnp.dot(s2, w2b_ref[...], preferred_element_type=jnp.float32)
            + b2_ref[...]
            + bias_ref[...]
        )
        out_n_ref[...] = jnp.maximum(h2, 0.0)


def kernel(X, E0, E1, alpha, W1, b1, W2, b2, bias):
    alpha2 = jnp.asarray(alpha, jnp.float32).reshape(1, 1)
    a, res = pl.pallas_call(
        _xdw_kernel,
        out_shape=[
            jax.ShapeDtypeStruct((N, F), jnp.float32),
            jax.ShapeDtypeStruct((N, F), jnp.float32),
        ],
    )(X, W1, b1.reshape(1, F), alpha2)

    grid = (N // RB, N // CB)
    out_e_flat, out_n = pl.pallas_call(
        _main_kernel,
        grid=grid,
        in_specs=[
            pl.BlockSpec((RB, CB), lambda i, j: (i, j)),
            pl.BlockSpec((RB, CB), lambda i, j: (i, j)),
            pl.BlockSpec((CB, F), lambda i, j: (j, 0)),
            pl.BlockSpec((RB, F), lambda i, j: (i, 0)),
            pl.BlockSpec((F, F), lambda i, j: (0, 0)),
            pl.BlockSpec((F, F), lambda i, j: (0, 0)),
            pl.BlockSpec((1, F), lambda i, j: (0, 0)),
            pl.BlockSpec((1, F), lambda i, j: (0, 0)),
        ],
        out_specs=[
            pl.BlockSpec((RB, 2 * CB), lambda i, j: (i, j)),
            pl.BlockSpec((RB, F), lambda i, j: (i, 0)),
        ],
        out_shape=[
            jax.ShapeDtypeStruct((N, 2 * N), jnp.float32),
            jax.ShapeDtypeStruct((N, F), jnp.float32),
        ],
        scratch_shapes=[
            pltpu.VMEM((RB, F), jnp.float32),
            pltpu.VMEM((RB, F), jnp.float32),
        ],
    )(E0, E1, a, res, W2[:F], W2[F:], b2.reshape(1, F), bias.reshape(1, F))

    out_E = out_e_flat.reshape(N, N // 128, 2, 128).swapaxes(2, 3).reshape(N, N, 2)
    return (out_n, out_E)
